# Initial kernel scaffold; baseline (speedup 1.0000x reference)
import jax
import jax.numpy as jnp
from jax.experimental import pallas as pl


def kernel(x_seq, h0, w_x, w_h, b):
    raise NotImplementedError("write your pallas kernel here")


# speedup vs baseline: 1.8853x; 1.8853x over previous
import functools

import jax
import jax.numpy as jnp
from jax.experimental import pallas as pl
from jax.experimental.pallas import tpu as pltpu


def _scan_kernel(x_ref, h0_ref, xl_ref, wah_ref, whh_ref, wx_ref, wh_ref,
                 b_ref, out_ref, hid_ref, grev, acc, *, T, IP, H, KC):
    k = pl.program_id(1)
    nk = pl.num_programs(1)

    @pl.when(k == 0)
    def _():
        w = whh_ref[...]
        grev[(T - 2) * IP:(T - 1) * IP, :] = wah_ref[...]
        grev[(T - 1) * IP:T * IP, :] = jnp.zeros((IP, H), jnp.float32)
        wm = w
        racc = w
        m = 1
        while m < T // 2:
            lo = (T - 1 - 2 * m) * IP
            mid = (T - 1 - m) * IP
            grev[lo:mid, :] = jnp.dot(
                grev[mid:(T - 1) * IP, :], wm,
                preferred_element_type=jnp.float32)
            wm = jnp.dot(wm, wm, preferred_element_type=jnp.float32)
            racc = jnp.dot(racc, wm, preferred_element_type=jnp.float32)
            m *= 2
        grev[0:(T // 2 - 1) * IP, :] = jnp.dot(
            grev[(T // 2) * IP:(T - 1) * IP, :], wm,
            preferred_element_type=jnp.float32)
        acc[...] = jnp.dot(h0_ref[...], racc,
                           preferred_element_type=jnp.float32)

    acc[...] += jnp.dot(x_ref[...], grev[pl.ds(k * KC, KC), :],
                        preferred_element_type=jnp.float32)

    @pl.when(k == nk - 1)
    def _():
        y = (jnp.dot(xl_ref[...], wx_ref[...],
                     preferred_element_type=jnp.float32)
             + b_ref[...]
             + jnp.dot(acc[...], wh_ref[...],
                       preferred_element_type=jnp.float32))
        hid_ref[...] = y[:, :H]
        logits = y[:, H:]
        mx = jnp.max(logits, axis=-1, keepdims=True)
        sh = logits - mx
        lse = jnp.log(jnp.sum(jnp.exp(sh), axis=-1, keepdims=True))
        out_ref[...] = (sh - lse).astype(out_ref.dtype)


def kernel(x_seq, h0, w_x, w_h, b):
    T, B, I = x_seq.shape
    H = h0.shape[1]
    HO = w_h.shape[1]
    O_pad = HO - H
    IP = 64
    assert I + 1 <= IP and T >= 4 and (T & (T - 1)) == 0

    f32 = jnp.float32
    parts = [x_seq, jnp.ones((T, B, 1), f32)]
    if IP - I - 1:
        parts.append(jnp.zeros((T, B, IP - I - 1), f32))
    x_flat = jnp.concatenate(parts, -1).transpose(1, 0, 2).reshape(B, T * IP)
    x_last = x_seq[T - 1]
    w_aug_h = (jnp.zeros((IP, H), f32)
               .at[:I].set(w_x[:, :H])
               .at[I].set(b[0, :H]))
    w_hh = w_h[:, :H]

    Bc = min(256, B)
    KC = min(2048, T * IP)
    assert B % Bc == 0 and (T * IP) % KC == 0 and KC % IP == 0

    out, hid = pl.pallas_call(
        functools.partial(_scan_kernel, T=T, IP=IP, H=H, KC=KC),
        out_shape=(jax.ShapeDtypeStruct((B, O_pad), f32),
                   jax.ShapeDtypeStruct((B, H), f32)),
        grid_spec=pltpu.PrefetchScalarGridSpec(
            num_scalar_prefetch=0,
            grid=(B // Bc, (T * IP) // KC),
            in_specs=[
                pl.BlockSpec((Bc, KC), lambda bb, k: (bb, k)),
                pl.BlockSpec((Bc, H), lambda bb, k: (bb, 0)),
                pl.BlockSpec((Bc, I), lambda bb, k: (bb, 0)),
                pl.BlockSpec((IP, H), lambda bb, k: (0, 0)),
                pl.BlockSpec((H, H), lambda bb, k: (0, 0)),
                pl.BlockSpec((I, HO), lambda bb, k: (0, 0)),
                pl.BlockSpec((H, HO), lambda bb, k: (0, 0)),
                pl.BlockSpec((1, HO), lambda bb, k: (0, 0)),
            ],
            out_specs=(pl.BlockSpec((Bc, O_pad), lambda bb, k: (bb, 0)),
                       pl.BlockSpec((Bc, H), lambda bb, k: (bb, 0))),
            scratch_shapes=[pltpu.VMEM((T * IP, H), f32),
                            pltpu.VMEM((Bc, H), f32)],
        ),
        compiler_params=pltpu.CompilerParams(
            dimension_semantics=("parallel", "arbitrary")),
    )(x_flat, h0, x_last, w_aug_h, w_hh, w_x, w_h, b)
    return out, hid


# speedup vs baseline: 2.4800x; 1.3154x over previous
import functools

import jax
import jax.numpy as jnp
from jax import lax
from jax.experimental import pallas as pl
from jax.experimental.pallas import tpu as pltpu


def _scan_kernel(x_ref, h0_ref, wah_ref, whh_ref, wx_ref, wh_ref, b_ref,
                 out_ref, hid_ref, grev, acc, *, T, IP, I, H, Tc):
    k = pl.program_id(1)
    nk = pl.num_programs(1)

    @pl.when(k == 0)
    def _():
        w = whh_ref[...]
        grev[(T - 2) * IP:(T - 1) * IP, :] = wah_ref[...]
        grev[(T - 1) * IP:T * IP, :] = jnp.zeros((IP, H), jnp.float32)
        eye = (lax.broadcasted_iota(jnp.int32, (H, H), 0)
               == lax.broadcasted_iota(jnp.int32, (H, H), 1)
               ).astype(jnp.float32)
        wm = w
        racc = w
        sacc = eye
        m = 1
        while m < T // 2:
            lo = (T - 1 - 2 * m) * IP
            mid = (T - 1 - m) * IP
            grev[lo:mid, :] = jnp.dot(
                grev[mid:(T - 1) * IP, :], wm,
                preferred_element_type=jnp.float32)
            sacc = sacc + jnp.dot(sacc, wm,
                                  preferred_element_type=jnp.float32)
            wm = jnp.dot(wm, wm, preferred_element_type=jnp.float32)
            racc = jnp.dot(racc, wm, preferred_element_type=jnp.float32)
            m *= 2
        grev[0:(T // 2 - 1) * IP, :] = jnp.dot(
            grev[(T // 2) * IP:(T - 1) * IP, :], wm,
            preferred_element_type=jnp.float32)
        sacc = sacc + jnp.dot(sacc, wm, preferred_element_type=jnp.float32)
        bias_row = jnp.dot(b_ref[:, :H], sacc - racc,
                           preferred_element_type=jnp.float32)
        acc[...] = (jnp.dot(h0_ref[...], racc,
                            preferred_element_type=jnp.float32) + bias_row)

    a = acc[...]
    for s in range(Tc):
        g = grev[pl.ds((k * Tc + s) * IP, IP), :]
        a = a + jnp.dot(x_ref[s], g[:I, :],
                        preferred_element_type=jnp.float32)
    acc[...] = a

    @pl.when(k == nk - 1)
    def _():
        y = (jnp.dot(x_ref[Tc - 1], wx_ref[...],
                     preferred_element_type=jnp.float32)
             + b_ref[...]
             + jnp.dot(a, wh_ref[...], preferred_element_type=jnp.float32))
        hid_ref[...] = y[:, :H]
        logits = y[:, H:]
        mx = jnp.max(logits, axis=-1, keepdims=True)
        sh = logits - mx
        lse = jnp.log(jnp.sum(jnp.exp(sh), axis=-1, keepdims=True))
        out_ref[...] = (sh - lse).astype(out_ref.dtype)


def kernel(x_seq, h0, w_x, w_h, b):
    T, B, I = x_seq.shape
    H = h0.shape[1]
    HO = w_h.shape[1]
    O_pad = HO - H
    IP = 64
    assert I <= IP and T >= 4 and (T & (T - 1)) == 0

    f32 = jnp.float32
    w_aug_h = jnp.zeros((IP, H), f32).at[:I].set(w_x[:, :H])
    w_hh = w_h[:, :H]

    Bc = min(256, B)
    Tc = min(16, T)
    assert B % Bc == 0 and T % Tc == 0

    out, hid = pl.pallas_call(
        functools.partial(_scan_kernel, T=T, IP=IP, I=I, H=H, Tc=Tc),
        out_shape=(jax.ShapeDtypeStruct((B, O_pad), f32),
                   jax.ShapeDtypeStruct((B, H), f32)),
        grid_spec=pltpu.PrefetchScalarGridSpec(
            num_scalar_prefetch=0,
            grid=(B // Bc, T // Tc),
            in_specs=[
                pl.BlockSpec((Tc, Bc, I), lambda bb, k: (k, bb, 0)),
                pl.BlockSpec((Bc, H), lambda bb, k: (bb, 0)),
                pl.BlockSpec((IP, H), lambda bb, k: (0, 0)),
                pl.BlockSpec((H, H), lambda bb, k: (0, 0)),
                pl.BlockSpec((I, HO), lambda bb, k: (0, 0)),
                pl.BlockSpec((H, HO), lambda bb, k: (0, 0)),
                pl.BlockSpec((1, HO), lambda bb, k: (0, 0)),
            ],
            out_specs=(pl.BlockSpec((Bc, O_pad), lambda bb, k: (bb, 0)),
                       pl.BlockSpec((Bc, H), lambda bb, k: (bb, 0))),
            scratch_shapes=[pltpu.VMEM((T * IP, H), f32),
                            pltpu.VMEM((Bc, H), f32)],
        ),
        compiler_params=pltpu.CompilerParams(
            dimension_semantics=("parallel", "arbitrary")),
    )(x_seq, h0, w_aug_h, w_hh, w_x, w_h, b)
    return out, hid


# speedup vs baseline: 3.0163x; 1.2162x over previous
import functools

import jax
import jax.numpy as jnp
from jax import lax
from jax.experimental import pallas as pl
from jax.experimental.pallas import tpu as pltpu


def _scan_kernel(x_ref, h0_ref, wah_ref, whh_ref, wx_ref, wh_ref, b_ref,
                 out_ref, hid_ref, grev, acc, *, T, IP, I, H, Tc):
    k = pl.program_id(0)
    nk = pl.num_programs(0)

    @pl.when(k == 0)
    def _():
        w = whh_ref[...]
        grev[(T - 2) * IP:(T - 1) * IP, :] = wah_ref[...]
        grev[(T - 1) * IP:T * IP, :] = jnp.zeros((IP, H), jnp.float32)
        eye = (lax.broadcasted_iota(jnp.int32, (H, H), 0)
               == lax.broadcasted_iota(jnp.int32, (H, H), 1)
               ).astype(jnp.float32)
        wm = w
        racc = w
        sacc = eye
        m = 1
        while m < T // 2:
            lo = (T - 1 - 2 * m) * IP
            mid = (T - 1 - m) * IP
            grev[lo:mid, :] = jnp.dot(
                grev[mid:(T - 1) * IP, :], wm,
                preferred_element_type=jnp.float32)
            sacc = sacc + jnp.dot(sacc, wm,
                                  preferred_element_type=jnp.float32)
            wm = jnp.dot(wm, wm, preferred_element_type=jnp.float32)
            racc = jnp.dot(racc, wm, preferred_element_type=jnp.float32)
            m *= 2
        grev[0:(T // 2 - 1) * IP, :] = jnp.dot(
            grev[(T // 2) * IP:(T - 1) * IP, :], wm,
            preferred_element_type=jnp.float32)
        sacc = sacc + jnp.dot(sacc, wm, preferred_element_type=jnp.float32)
        bias_row = jnp.dot(b_ref[:, :H], sacc - racc,
                           preferred_element_type=jnp.float32)
        acc[...] = (jnp.dot(h0_ref[...], racc,
                            preferred_element_type=jnp.float32) + bias_row)

    zpad = jnp.zeros((x_ref.shape[1], IP - I), jnp.float32)
    a = acc[...]
    for s0 in range(0, Tc, 4):
        xs = jnp.concatenate(
            [x_ref[s0], zpad, x_ref[s0 + 1], zpad,
             x_ref[s0 + 2], zpad, x_ref[s0 + 3], zpad], axis=1)
        g = grev[pl.ds((k * Tc + s0) * IP, 4 * IP), :]
        a = a + jnp.dot(xs, g, preferred_element_type=jnp.float32)
    acc[...] = a

    @pl.when(k == nk - 1)
    def _():
        y = (jnp.dot(x_ref[Tc - 1], wx_ref[...],
                     preferred_element_type=jnp.float32)
             + b_ref[...]
             + jnp.dot(a, wh_ref[...], preferred_element_type=jnp.float32))
        hid_ref[...] = y[:, :H]
        logits = y[:, H:]
        mx = jnp.max(logits, axis=-1, keepdims=True)
        sh = logits - mx
        lse = jnp.log(jnp.sum(jnp.exp(sh), axis=-1, keepdims=True))
        out_ref[...] = (sh - lse).astype(out_ref.dtype)


def kernel(x_seq, h0, w_x, w_h, b):
    T, B, I = x_seq.shape
    H = h0.shape[1]
    HO = w_h.shape[1]
    O_pad = HO - H
    IP = 64
    assert I <= IP and T >= 4 and (T & (T - 1)) == 0

    f32 = jnp.float32
    w_aug_h = jnp.zeros((IP, H), f32).at[:I].set(w_x[:, :H])
    w_hh = w_h[:, :H]

    Tc = min(32, T)
    assert T % Tc == 0 and Tc % 4 == 0

    out, hid = pl.pallas_call(
        functools.partial(_scan_kernel, T=T, IP=IP, I=I, H=H, Tc=Tc),
        out_shape=(jax.ShapeDtypeStruct((B, O_pad), f32),
                   jax.ShapeDtypeStruct((B, H), f32)),
        grid_spec=pltpu.PrefetchScalarGridSpec(
            num_scalar_prefetch=0,
            grid=(T // Tc,),
            in_specs=[
                pl.BlockSpec((Tc, B, I), lambda k: (k, 0, 0)),
                pl.BlockSpec((B, H), lambda k: (0, 0)),
                pl.BlockSpec((IP, H), lambda k: (0, 0)),
                pl.BlockSpec((H, H), lambda k: (0, 0)),
                pl.BlockSpec((I, HO), lambda k: (0, 0)),
                pl.BlockSpec((H, HO), lambda k: (0, 0)),
                pl.BlockSpec((1, HO), lambda k: (0, 0)),
            ],
            out_specs=(pl.BlockSpec((B, O_pad), lambda k: (0, 0)),
                       pl.BlockSpec((B, H), lambda k: (0, 0))),
            scratch_shapes=[pltpu.VMEM((T * IP, H), f32),
                            pltpu.VMEM((B, H), f32)],
        ),
        compiler_params=pltpu.CompilerParams(
            dimension_semantics=("arbitrary",)),
    )(x_seq, h0, w_aug_h, w_hh, w_x, w_h, b)
    return out, hid
